# baseline (speedup 1.0000x reference)
import dataclasses
import functools

import jax
import jax.numpy as jnp
from jax import lax
from jax.experimental import pallas as pl
from jax.experimental.pallas import tpu as pltpu
from jax.experimental.pallas import tpu_sc as plsc

B = 1024
L = 200
EMB = 300
NUM_LABELS = 32768

NC = 2
NS = 16
NW = NC * NS
ROWS_PER_W = B // NW
NFULL = EMB // 16
TAIL_OFF = EMB - 16
EMB_P = 304

_vector_mesh = plsc.VectorSubcoreMesh(core_axis_name="c", subcore_axis_name="s")

_sc_params = pltpu.CompilerParams()
if "needs_layout_passes" in pltpu.CompilerParams.__dataclass_fields__:
    _sc_params = dataclasses.replace(_sc_params, needs_layout_passes=False)
if "use_tc_tiling_on_sc" in pltpu.CompilerParams.__dataclass_fields__:
    _sc_params = dataclasses.replace(_sc_params, use_tc_tiling_on_sc=False)


@functools.partial(
    pl.kernel,
    out_type=jax.ShapeDtypeStruct((B, EMB), jnp.float32),
    mesh=_vector_mesh,
    scratch_types=[
        pltpu.VMEM((ROWS_PER_W, L), jnp.int32),
        pltpu.VMEM((ROWS_PER_W, L), jnp.float32),
        pltpu.VMEM((L, EMB_P), jnp.float32),
        pltpu.VMEM((ROWS_PER_W, EMB), jnp.float32),
    ],
    compiler_params=_sc_params,
)
def _sc_embed(x_hbm, w_hbm, table_hbm, rep_hbm, idx_v, wv, gbuf, repst):
    wid = lax.axis_index("s") * NC + lax.axis_index("c")
    base = wid * ROWS_PER_W
    pltpu.sync_copy(x_hbm.at[pl.ds(base, ROWS_PER_W)], idx_v)
    pltpu.sync_copy(w_hbm.at[pl.ds(base, ROWS_PER_W)], wv)

    @pl.loop(0, ROWS_PER_W)
    def _row(b):
        pltpu.sync_copy(table_hbm.at[idx_v.at[b]], gbuf)

        def body(l, accs):
            wl = plsc.load_gather(
                wv,
                [jnp.full((16,), b, jnp.int32), jnp.full((16,), l, jnp.int32)],
            )
            new = [accs[k] + wl * gbuf[l, pl.ds(k * 16, 16)] for k in range(NFULL)]
            new.append(accs[NFULL] + wl * gbuf[l, pl.ds(TAIL_OFF, 16)])
            return tuple(new)

        init = tuple(jnp.zeros((16,), jnp.float32) for _ in range(NFULL + 1))
        accs = lax.fori_loop(0, L, body, init)
        for k in range(NFULL):
            repst[b, pl.ds(k * 16, 16)] = accs[k]
        repst[b, pl.ds(TAIL_OFF, 16)] = accs[NFULL]

    pltpu.sync_copy(repst, rep_hbm.at[pl.ds(base, ROWS_PER_W)])


BN = 2048


def _tc_body(rep_ref, w_ref, b_ref, out_ref):
    r = jnp.maximum(rep_ref[...], 0.0).astype(jnp.bfloat16)
    w = w_ref[...].astype(jnp.bfloat16)
    acc = lax.dot_general(
        r, w, (((1,), (1,)), ((), ())), preferred_element_type=jnp.float32
    )
    out_ref[...] = acc + b_ref[...]


def _tc_classify(rep, clf_W, clf_b):
    return pl.pallas_call(
        _tc_body,
        grid=(NUM_LABELS // BN,),
        in_specs=[
            pl.BlockSpec((B, EMB), lambda i: (0, 0)),
            pl.BlockSpec((BN, EMB), lambda i: (i, 0)),
            pl.BlockSpec((1, BN), lambda i: (0, i)),
        ],
        out_specs=pl.BlockSpec((B, BN), lambda i: (0, i)),
        out_shape=jax.ShapeDtypeStruct((B, NUM_LABELS), jnp.float32),
    )(rep, clf_W, clf_b.reshape(1, NUM_LABELS))


def kernel(X, X_w, emb_table, clf_W, clf_b):
    X = X.astype(jnp.int32)
    table_p = jnp.pad(emb_table, ((0, 0), (0, EMB_P - EMB)))
    rep = _sc_embed(X, X_w, table_p)
    return _tc_classify(rep, clf_W, clf_b)


# speedup vs baseline: 1.5486x; 1.5486x over previous
import dataclasses
import functools

import jax
import jax.numpy as jnp
from jax import lax
from jax.experimental import pallas as pl
from jax.experimental.pallas import tpu as pltpu
from jax.experimental.pallas import tpu_sc as plsc

B = 1024
L = 200
EMB = 300
NUM_LABELS = 32768

NC = 2
NS = 16
NW = NC * NS
ROWS_PER_W = B // NW
NFULL = EMB // 16
TAIL_OFF = EMB - 16
EMB_P = 304

_vector_mesh = plsc.VectorSubcoreMesh(core_axis_name="c", subcore_axis_name="s")

_sc_params = pltpu.CompilerParams()
if "needs_layout_passes" in pltpu.CompilerParams.__dataclass_fields__:
    _sc_params = dataclasses.replace(_sc_params, needs_layout_passes=False)
if "use_tc_tiling_on_sc" in pltpu.CompilerParams.__dataclass_fields__:
    _sc_params = dataclasses.replace(_sc_params, use_tc_tiling_on_sc=False)


@functools.partial(
    pl.kernel,
    out_type=jax.ShapeDtypeStruct((B, EMB), jnp.float32),
    mesh=_vector_mesh,
    scratch_types=[
        pltpu.VMEM((ROWS_PER_W, L), jnp.int32),
        pltpu.VMEM((ROWS_PER_W, L), jnp.float32),
        pltpu.VMEM((L, EMB_P), jnp.float32),
        pltpu.VMEM((ROWS_PER_W, EMB), jnp.float32),
    ],
    compiler_params=_sc_params,
)
def _sc_embed(x_hbm, w_hbm, table_hbm, rep_hbm, idx_v, wv, gbuf, repst):
    wid = lax.axis_index("s") * NC + lax.axis_index("c")
    base = wid * ROWS_PER_W
    pltpu.sync_copy(x_hbm.at[pl.ds(base, ROWS_PER_W)], idx_v)
    pltpu.sync_copy(w_hbm.at[pl.ds(base, ROWS_PER_W)], wv)

    @pl.loop(0, ROWS_PER_W)
    def _row(b):
        pltpu.sync_copy(table_hbm.at[idx_v.at[b]], gbuf)

        def body(l, accs):
            wl = plsc.load_gather(
                wv,
                [jnp.full((16,), b, jnp.int32), jnp.full((16,), l, jnp.int32)],
            )
            new = [accs[k] + wl * gbuf[l, pl.ds(k * 16, 16)] for k in range(NFULL)]
            new.append(accs[NFULL] + wl * gbuf[l, pl.ds(TAIL_OFF, 16)])
            return tuple(new)

        init = tuple(jnp.zeros((16,), jnp.float32) for _ in range(NFULL + 1))
        accs = lax.fori_loop(0, L, body, init)
        for k in range(NFULL):
            repst[b, pl.ds(k * 16, 16)] = accs[k]
        repst[b, pl.ds(TAIL_OFF, 16)] = accs[NFULL]

    pltpu.sync_copy(repst, rep_hbm.at[pl.ds(base, ROWS_PER_W)])


VOCAB = 100001
PAD_BR = 8192


def _pad_body(t_ref, o_ref):
    o_ref[:, :EMB] = t_ref[...]


def _pad_table(emb_table):
    grid = ((VOCAB + PAD_BR - 1) // PAD_BR,)
    return pl.pallas_call(
        _pad_body,
        grid=grid,
        in_specs=[pl.BlockSpec((PAD_BR, EMB), lambda i: (i, 0))],
        out_specs=pl.BlockSpec((PAD_BR, EMB_P), lambda i: (i, 0)),
        out_shape=jax.ShapeDtypeStruct((VOCAB, EMB_P), jnp.float32),
    )(emb_table)


BN = 2048


def _tc_body(rep_ref, w_ref, b_ref, out_ref):
    r = jnp.maximum(rep_ref[...], 0.0).astype(jnp.bfloat16)
    w = w_ref[...].astype(jnp.bfloat16)
    acc = lax.dot_general(
        r, w, (((1,), (1,)), ((), ())), preferred_element_type=jnp.float32
    )
    out_ref[...] = acc + b_ref[...]


def _tc_classify(rep, clf_W, clf_b):
    return pl.pallas_call(
        _tc_body,
        grid=(NUM_LABELS // BN,),
        in_specs=[
            pl.BlockSpec((B, EMB), lambda i: (0, 0)),
            pl.BlockSpec((BN, EMB), lambda i: (i, 0)),
            pl.BlockSpec((1, BN), lambda i: (0, i)),
        ],
        out_specs=pl.BlockSpec((B, BN), lambda i: (0, i)),
        out_shape=jax.ShapeDtypeStruct((B, NUM_LABELS), jnp.float32),
    )(rep, clf_W, clf_b.reshape(1, NUM_LABELS))


def kernel(X, X_w, emb_table, clf_W, clf_b):
    X = X.astype(jnp.int32)
    table_p = _pad_table(emb_table)
    rep = _sc_embed(X, X_w, table_p)
    return _tc_classify(rep, clf_W, clf_b)


# speedup vs baseline: 2.7648x; 1.7854x over previous
import dataclasses
import functools

import jax
import jax.numpy as jnp
from jax import lax
from jax.experimental import pallas as pl
from jax.experimental.pallas import tpu as pltpu
from jax.experimental.pallas import tpu_sc as plsc

B = 1024
L = 200
EMB = 300
NUM_LABELS = 32768
VOCAB = 100001

NC = 2
NS = 16
NW = NC * NS
ROWS_PER_W = B // NW
NFULL = EMB // 16
LH0 = 104
LH1 = L - LH0

_vector_mesh = plsc.VectorSubcoreMesh(core_axis_name="c", subcore_axis_name="s")

_sc_params = pltpu.CompilerParams()
if "needs_layout_passes" in pltpu.CompilerParams.__dataclass_fields__:
    _sc_params = dataclasses.replace(_sc_params, needs_layout_passes=False)
if "use_tc_tiling_on_sc" in pltpu.CompilerParams.__dataclass_fields__:
    _sc_params = dataclasses.replace(_sc_params, use_tc_tiling_on_sc=False)


RB = 1024


S2_OFF = EMB - 128


def _repack_body(t_ref, o0_ref, o1_ref, o2_ref):
    v = jnp.transpose(t_ref[...])
    o0_ref[...] = v[:, 0:128]
    o1_ref[...] = v[:, 128:256]
    o2_ref[...] = v[:, S2_OFF:S2_OFF + 128]


def _repack(emb_table_t):
    grid = ((VOCAB + RB - 1) // RB,)
    out = jax.ShapeDtypeStruct((VOCAB, 128), jnp.float32)
    return pl.pallas_call(
        _repack_body,
        grid=grid,
        in_specs=[pl.BlockSpec((EMB + 4, RB), lambda i: (0, i))],
        out_specs=[pl.BlockSpec((RB, 128), lambda i: (i, 0))] * 3,
        out_shape=[out, out, out],
    )(emb_table_t)


def _splat16(ref, i, j):
    return plsc.load_gather(
        ref, [jnp.full((16,), i, jnp.int32), jnp.full((16,), j, jnp.int32)]
    )


@functools.partial(
    pl.kernel,
    out_type=jax.ShapeDtypeStruct((B, EMB), jnp.float32),
    mesh=_vector_mesh,
    scratch_types=[
        pltpu.VMEM((ROWS_PER_W, L), jnp.int32),
        pltpu.VMEM((ROWS_PER_W, L), jnp.float32),
        pltpu.VMEM((LH0, 128), jnp.float32),
        pltpu.VMEM((LH0, 128), jnp.float32),
        pltpu.VMEM((LH0, 128), jnp.float32),
        pltpu.VMEM((LH1, 128), jnp.float32),
        pltpu.VMEM((LH1, 128), jnp.float32),
        pltpu.VMEM((LH1, 128), jnp.float32),
        pltpu.VMEM((ROWS_PER_W, EMB), jnp.float32),
        pltpu.SemaphoreType.DMA,
        pltpu.SemaphoreType.DMA,
    ],
    compiler_params=_sc_params,
)
def _sc_embed(x_hbm, w_hbm, t0_hbm, t1_hbm, t2_hbm, rep_hbm,
              xv, wv, a0, a1, a2, b0, b1, b2, repst, semA, semB):
    wid = lax.axis_index("s") * NC + lax.axis_index("c")
    base = wid * ROWS_PER_W
    pltpu.sync_copy(x_hbm.at[pl.ds(base, ROWS_PER_W)], xv)
    pltpu.sync_copy(w_hbm.at[pl.ds(base, ROWS_PER_W)], wv)

    def start(b, off, n, g0, g1, g2, sem):
        idx = xv.at[b, pl.ds(off, n)]
        pltpu.make_async_copy(t0_hbm.at[idx], g0, sem).start()
        pltpu.make_async_copy(t1_hbm.at[idx], g1, sem).start()
        pltpu.make_async_copy(t2_hbm.at[idx], g2, sem).start()

    def wait(b, off, n, g0, g1, g2, sem):
        idx = xv.at[b, pl.ds(off, n)]
        pltpu.make_async_copy(t0_hbm.at[idx], g0, sem).wait()
        pltpu.make_async_copy(t1_hbm.at[idx], g1, sem).wait()
        pltpu.make_async_copy(t2_hbm.at[idx], g2, sem).wait()

    def accumulate(g0, g1, g2, b, off, n, accs):
        def lbody(ll, accs):
            l = off + ll
            w_vec = _splat16(wv, b, l)
            new = []
            for k in range(8):
                new.append(accs[k] + w_vec * g0[ll, pl.ds(k * 16, 16)])
            for k in range(8, 16):
                new.append(accs[k] + w_vec * g1[ll, pl.ds((k - 8) * 16, 16)])
            for k in range(16, NFULL):
                new.append(
                    accs[k] + w_vec * g2[ll, pl.ds(k * 16 - S2_OFF, 16)]
                )
            new.append(accs[NFULL] + w_vec * g2[ll, pl.ds(EMB - 16 - S2_OFF, 16)])
            return tuple(new)

        return lax.fori_loop(0, n, lbody, accs)

    zeros = tuple(jnp.zeros((16,), jnp.float32) for _ in range(NFULL + 1))

    start(0, 0, LH0, a0, a1, a2, semA)

    @pl.loop(0, ROWS_PER_W)
    def _row(b):
        start(b, LH0, LH1, b0, b1, b2, semB)
        wait(b, 0, LH0, a0, a1, a2, semA)
        accs = accumulate(a0, a1, a2, b, 0, LH0, zeros)

        @pl.when(b < ROWS_PER_W - 1)
        def _():
            start(b + 1, 0, LH0, a0, a1, a2, semA)

        wait(b, LH0, LH1, b0, b1, b2, semB)
        accs = accumulate(b0, b1, b2, b, LH0, LH1, accs)

        for k in range(NFULL):
            repst[b, pl.ds(k * 16, 16)] = accs[k]
        repst[b, pl.ds(EMB - 16, 16)] = accs[NFULL]

    pltpu.sync_copy(repst, rep_hbm.at[pl.ds(base, ROWS_PER_W)])


BN = 2048


def _tc_body(rep_ref, w_ref, b_ref, out_ref):
    r = jnp.maximum(rep_ref[...], 0.0).astype(jnp.bfloat16)
    w = w_ref[...].astype(jnp.bfloat16)
    acc = lax.dot_general(
        r, w, (((1,), (0,)), ((), ())), preferred_element_type=jnp.float32
    )
    out_ref[...] = acc + b_ref[...]


def _tc_classify(rep, clf_W, clf_b):
    return pl.pallas_call(
        _tc_body,
        grid=(NUM_LABELS // BN,),
        in_specs=[
            pl.BlockSpec((B, EMB), lambda i: (0, 0)),
            pl.BlockSpec((EMB, BN), lambda i: (0, i)),
            pl.BlockSpec((1, BN), lambda i: (0, i)),
        ],
        out_specs=pl.BlockSpec((B, BN), lambda i: (0, i)),
        out_shape=jax.ShapeDtypeStruct((B, NUM_LABELS), jnp.float32),
    )(rep, clf_W.T, clf_b.reshape(1, NUM_LABELS))


def kernel(X, X_w, emb_table, clf_W, clf_b):
    X = X.astype(jnp.int32)
    t0, t1, t2 = _repack(emb_table.T)
    rep = _sc_embed(X, X_w, t0, t1, t2)
    return _tc_classify(rep, clf_W, clf_b)
